# baseline (speedup 1.0000x reference)
import functools
import math

import jax
import jax.numpy as jnp
from jax import lax
from jax.experimental import pallas as pl
from jax.experimental.pallas import tpu as pltpu
from jax.experimental.pallas import tpu_sc as plsc

D = 128
SCALE = math.sqrt(float(D))

NC = 2
NS = 16
NW = NC * NS
C = 128
LANES = 16


def _scale_rows(src, dst, dst_off):
    def body(i, _):
        r = i * 2
        for rr in range(2):
            for l in range(D // LANES):
                off = l * LANES
                dst[dst_off + r + rr, pl.ds(off, LANES)] = (
                    src[r + rr, pl.ds(off, LANES)] * SCALE
                )
        return 0

    lax.fori_loop(0, C // 2, body, 0)


def _make_emb(B, NCH):
    NP = NCH // 2
    mesh = plsc.VectorSubcoreMesh(core_axis_name="c", subcore_axis_name="s")

    @functools.partial(
        pl.kernel,
        mesh=mesh,
        out_type=jax.ShapeDtypeStruct((B, D), jnp.float32),
        scratch_types=[
            pltpu.VMEM((NCH * C,), jnp.int32),
            pltpu.VMEM((2, C, D), jnp.float32),
            pltpu.VMEM((2, 2 * C, D), jnp.float32),
            pltpu.SemaphoreType.DMA,
            pltpu.SemaphoreType.DMA,
            pltpu.SemaphoreType.DMA,
            pltpu.SemaphoreType.DMA,
        ],
    )
    def emb(table_hbm, idx_hbm, out_hbm, idx_v, g_ref, s_ref,
            gs0, gs1, ss0, ss1):
        cid = lax.axis_index("c")
        sid = lax.axis_index("s")
        wid = sid * NC + cid
        base_row = wid * (NCH * C)

        pltpu.sync_copy(idx_hbm.at[wid], idx_v)

        gsems = (gs0, gs1)
        ssems = (ss0, ss1)

        def gather_start(c, b):
            pltpu.make_async_copy(
                table_hbm.at[idx_v.at[pl.ds(c * C, C)]], g_ref.at[b], gsems[b]
            ).start()

        def gather_wait(c, b):
            pltpu.make_async_copy(
                table_hbm.at[idx_v.at[pl.ds(c * C, C)]], g_ref.at[b], gsems[b]
            ).wait()

        def scatter_start(p, sp):
            pltpu.make_async_copy(
                s_ref.at[sp],
                out_hbm.at[pl.ds(base_row + p * 2 * C, 2 * C)],
                ssems[sp],
            ).start()

        def scatter_wait(p, sp):
            pltpu.make_async_copy(
                s_ref.at[sp],
                out_hbm.at[pl.ds(base_row + p * 2 * C, 2 * C)],
                ssems[sp],
            ).wait()

        def pair_body(p, sp, first, issue_next=True):
            gather_wait(2 * p, 0)
            if not first:
                scatter_wait(p - 2, sp)
            _scale_rows(g_ref.at[0], s_ref.at[sp], 0)
            if issue_next:
                gather_start(2 * p + 2, 0)
            gather_wait(2 * p + 1, 1)
            _scale_rows(g_ref.at[1], s_ref.at[sp], C)
            if issue_next:
                gather_start(2 * p + 3, 1)
            scatter_start(p, sp)

        gather_start(0, 0)
        gather_start(1, 1)
        pair_body(0, 0, True)
        pair_body(1, 1, True)

        def main(p, _):
            pair_body(p, 0, False)
            pair_body(p + 1, 1, False)
            return 0

        n_main_pairs = NP - 2 - 1
        first_epi = 2 + (n_main_pairs // 2) * 2

        def main_loop(gi, _):
            p = 2 + gi * 2
            pair_body(p, 0, False)
            pair_body(p + 1, 1, False)
            return 0

        lax.fori_loop(0, n_main_pairs // 2, main_loop, 0)

        for p in range(first_epi, NP):
            pair_body(p, p % 2, False, issue_next=(2 * p + 3 < NCH))

        scatter_wait(NP - 2, (NP - 2) % 2)
        scatter_wait(NP - 1, (NP - 1) % 2)

    return emb


def kernel(tokens, table):
    n, t = tokens.shape
    B = n * t
    NCH = B // (NW * C)
    idx = tokens.reshape(-1).astype(jnp.int32).reshape(NW, NCH * C)
    out = _make_emb(B, NCH)(table, idx)
    return out.reshape(n, t, D)


# speedup vs baseline: 1.0106x; 1.0106x over previous
import functools
import math

import jax
import jax.numpy as jnp
from jax import lax
from jax.experimental import pallas as pl
from jax.experimental.pallas import tpu as pltpu
from jax.experimental.pallas import tpu_sc as plsc

D = 128
SCALE = math.sqrt(float(D))

NC = 2
NS = 16
NW = NC * NS
C = 128
NBUF = 2
LANES = 16


def _scale_rows(src, dst):
    def body(r, _):
        for l in range(D // LANES):
            off = l * LANES
            dst[r, pl.ds(off, LANES)] = src[r, pl.ds(off, LANES)] * SCALE
        return 0

    lax.fori_loop(0, C, body, 0)


def _make_emb(B, NCH):
    NG = NCH // NBUF
    mesh = plsc.VectorSubcoreMesh(core_axis_name="c", subcore_axis_name="s")

    @functools.partial(
        pl.kernel,
        mesh=mesh,
        out_type=jax.ShapeDtypeStruct((B, D), jnp.float32),
        scratch_types=[
            pltpu.VMEM((NCH * C,), jnp.int32),
            pltpu.VMEM((NBUF, C, D), jnp.float32),
            pltpu.VMEM((NBUF, C, D), jnp.float32),
            pltpu.SemaphoreType.DMA,
            pltpu.SemaphoreType.DMA,
            pltpu.SemaphoreType.DMA,
            pltpu.SemaphoreType.DMA,
        ],
    )
    def emb(table_hbm, idx_hbm, out_hbm, idx_v, g_ref, s_ref, gs0, gs1, ss0, ss1):
        cid = lax.axis_index("c")
        sid = lax.axis_index("s")
        wid = sid * NC + cid
        base_row = wid * (NCH * C)

        pltpu.sync_copy(idx_hbm.at[wid], idx_v)

        gsems = (gs0, gs1)
        ssems = (ss0, ss1)

        def gather_start(c, b):
            pltpu.make_async_copy(
                table_hbm.at[idx_v.at[pl.ds(c * C, C)]], g_ref.at[b], gsems[b]
            ).start()

        def gather_wait(c, b):
            pltpu.make_async_copy(
                table_hbm.at[idx_v.at[pl.ds(c * C, C)]], g_ref.at[b], gsems[b]
            ).wait()

        def scatter_start(c, b):
            pltpu.make_async_copy(
                s_ref.at[b], out_hbm.at[pl.ds(base_row + c * C, C)], ssems[b]
            ).start()

        def scatter_wait(c, b):
            pltpu.make_async_copy(
                s_ref.at[b], out_hbm.at[pl.ds(base_row + c * C, C)], ssems[b]
            ).wait()

        for b in range(NBUF):
            gather_start(b, b)

        def main(gi, _):
            for b in range(NBUF):
                c = gi * NBUF + b
                gather_wait(c, b)
                pl.when(gi >= 1)(lambda: scatter_wait(c - NBUF, b))
                _scale_rows(g_ref.at[b], s_ref.at[b])
                scatter_start(c, b)
                pl.when(gi <= NG - 2)(lambda: gather_start(c + NBUF, b))
            return 0

        lax.fori_loop(0, NG, main, 0)

        for b in range(NBUF):
            scatter_wait(NCH - NBUF + b, b)

    return emb


def kernel(tokens, table):
    n, t = tokens.shape
    B = n * t
    NCH = B // (NW * C)
    idx = tokens.reshape(-1).astype(jnp.int32).reshape(NW, NCH * C)
    out = _make_emb(B, NCH)(table, idx)
    return out.reshape(n, t, D)
